# baseline (speedup 1.0000x reference)
import functools
import jax
import jax.numpy as jnp
from jax import lax
from jax.experimental import pallas as pl
from jax.experimental.pallas import tpu as pltpu
from jax.experimental.pallas import tpu_sc as plsc

N_NODES = 10000
N_EDGES = 160000
D_FEAT = 256
NHID = 128
N_CLASSES = 2
N_GRAPHS = 128

NC, NS = 2, 16
HALF = NHID // 2
CHUNK = 128
CHUNKS_PER_TILE = 80
N_CHUNKS = NS * CHUNKS_PER_TILE
PAD_EDGES = N_CHUNKS * CHUNK - N_EDGES
ACC_ROWS = 10240
ROWS_PER_TILE = ACC_ROWS // NS
DUMMY_ROW = N_NODES
CNT_W = 16


def _lin_body(x_ref, wl_ref, wr_ref, b1_ref, y0_ref, y1_ref, z_ref):
    x = x_ref[...]
    y = jnp.dot(x, wl_ref[...], preferred_element_type=jnp.float32)
    y0_ref[...] = y[:, :HALF]
    y1_ref[...] = y[:, HALF:]
    z_ref[...] = jnp.dot(x, wr_ref[...], preferred_element_type=jnp.float32) + b1_ref[...]


def _linear(x, W_l, W_r, b1):
    blk = 1000
    return pl.pallas_call(
        _lin_body,
        grid=(N_NODES // blk,),
        in_specs=[
            pl.BlockSpec((blk, D_FEAT), lambda i: (i, 0)),
            pl.BlockSpec((D_FEAT, NHID), lambda i: (0, 0)),
            pl.BlockSpec((D_FEAT, NHID), lambda i: (0, 0)),
            pl.BlockSpec((1, NHID), lambda i: (0, 0)),
        ],
        out_specs=[
            pl.BlockSpec((blk, HALF), lambda i: (i, 0)),
            pl.BlockSpec((blk, HALF), lambda i: (i, 0)),
            pl.BlockSpec((blk, NHID), lambda i: (i, 0)),
        ],
        out_shape=[
            jax.ShapeDtypeStruct((N_NODES, HALF), jnp.float32),
            jax.ShapeDtypeStruct((N_NODES, HALF), jnp.float32),
            jax.ShapeDtypeStruct((N_NODES, NHID), jnp.float32),
        ],
    )(x, W_l, W_r, b1.reshape(1, NHID))


def _sc_body(y0_hbm, y1_hbm, src_hbm, dst_hbm, zrow_hbm, zcnt_hbm, ones_hbm,
             agg0_out, agg1_out, cnt_out,
             src_v, dst_v, rows_a, rows_b, rows_c, rows_d, ones_v,
             acc_sh, cnt_sh,
             gsem_a, gsem_b, gsem_c, gsem_d,
             ssem_a, ssem_b, ssem_c, ssem_d, sem_o, sem_p):
    c = lax.axis_index("c")
    s = lax.axis_index("s")
    r0 = pl.multiple_of(s * ROWS_PER_TILE, 8)

    c0 = pl.multiple_of(s * CHUNKS_PER_TILE, 8)
    prologue = [
        (zrow_hbm, acc_sh.at[pl.ds(r0, ROWS_PER_TILE)]),
        (zcnt_hbm, cnt_sh.at[pl.ds(r0, ROWS_PER_TILE)]),
        (ones_hbm, ones_v),
        (src_hbm.at[pl.ds(c0, CHUNKS_PER_TILE)], src_v),
        (dst_hbm.at[pl.ds(c0, CHUNKS_PER_TILE)], dst_v),
    ]
    for src_ref, dst_ref in prologue:
        pltpu.async_copy(src_ref, dst_ref, sem_p)
    for src_ref, dst_ref in prologue:
        pltpu.make_async_copy(src_ref, dst_ref, sem_p).wait()

    plsc.subcore_barrier()

    def edge_loop(y_hbm, cnt_lo, cnt_hi):
        bufs = (rows_a, rows_b, rows_c, rows_d)
        gsems = (gsem_a, gsem_b, gsem_c, gsem_d)
        ssems = (ssem_a, ssem_b, ssem_c, ssem_d)
        nsuper = CHUNKS_PER_TILE // 4

        def wait_gather(b):
            pltpu.make_async_copy(y_hbm.at[pl.ds(0, CHUNK)], bufs[b],
                                  gsems[b]).wait()

        def wait_scatter(b):
            pltpu.make_async_copy(bufs[b], acc_sh.at[dst_v.at[0]],
                                  ssems[b]).wait()

        def wait_ones():
            pltpu.make_async_copy(ones_v, cnt_sh.at[dst_v.at[0]],
                                  sem_o).wait()

        for b in range(4):
            pltpu.async_copy(y_hbm.at[src_v.at[b]], bufs[b], gsems[b])

        def super_body(j, carry):
            k0 = 4 * j

            @pl.when(jnp.logical_and(j > cnt_lo, j < cnt_hi))
            def _():
                for _b in range(4):
                    wait_ones()

            for b in range(4):
                wait_gather(b)
                pltpu.async_copy(bufs[b], acc_sh.at[dst_v.at[k0 + b]],
                                 ssems[b], add=True)

                @pl.when(jnp.logical_and(j >= cnt_lo, j < cnt_hi))
                def _():
                    pltpu.async_copy(ones_v, cnt_sh.at[dst_v.at[k0 + b]],
                                     sem_o, add=True)

            @pl.when(j < nsuper - 1)
            def _():
                for b in range(4):
                    wait_scatter(b)
                    pltpu.async_copy(y_hbm.at[src_v.at[k0 + 4 + b]],
                                     bufs[b], gsems[b])
            return carry

        lax.fori_loop(0, nsuper, super_body, 0)
        for b in range(4):
            wait_scatter(b)
        if cnt_hi > cnt_lo:
            for _b in range(4):
                wait_ones()

    nsuper = CHUNKS_PER_TILE // 4

    @pl.when(c == 0)
    def _():
        edge_loop(y0_hbm, 0, nsuper)

    @pl.when(c == 1)
    def _():
        edge_loop(y1_hbm, 0, 0)

    plsc.subcore_barrier()

    nvalid = N_NODES - (NS - 1) * ROWS_PER_TILE

    def writeback(nrows):
        @pl.when(c == 0)
        def _():
            pltpu.sync_copy(acc_sh.at[pl.ds(r0, nrows)],
                            agg0_out.at[pl.ds(r0, nrows)])
            pltpu.sync_copy(cnt_sh.at[pl.ds(r0, nrows)],
                            cnt_out.at[pl.ds(r0, nrows)])

        @pl.when(c == 1)
        def _():
            pltpu.sync_copy(acc_sh.at[pl.ds(r0, nrows)],
                            agg1_out.at[pl.ds(r0, nrows)])

    @pl.when(s < NS - 1)
    def _():
        writeback(ROWS_PER_TILE)

    @pl.when(s == NS - 1)
    def _():
        writeback(nvalid)


@functools.cache
def _sc_aggregate_kernel():
    mesh = plsc.VectorSubcoreMesh(
        core_axis_name="c", subcore_axis_name="s",
        num_cores=NC, num_subcores=NS)
    return pl.kernel(
        _sc_body,
        out_type=[
            jax.ShapeDtypeStruct((N_NODES, HALF), jnp.float32),
            jax.ShapeDtypeStruct((N_NODES, HALF), jnp.float32),
            jax.ShapeDtypeStruct((N_NODES, CNT_W), jnp.float32),
        ],
        mesh=mesh,
        compiler_params=pltpu.CompilerParams(use_tc_tiling_on_sc=False),
        scratch_types=[
            pltpu.VMEM((CHUNKS_PER_TILE, CHUNK), jnp.int32),
            pltpu.VMEM((CHUNKS_PER_TILE, CHUNK), jnp.int32),
            pltpu.VMEM((CHUNK, HALF), jnp.float32),
            pltpu.VMEM((CHUNK, HALF), jnp.float32),
            pltpu.VMEM((CHUNK, HALF), jnp.float32),
            pltpu.VMEM((CHUNK, HALF), jnp.float32),
            pltpu.VMEM((CHUNK, CNT_W), jnp.float32),
            pltpu.VMEM_SHARED((ACC_ROWS, HALF), jnp.float32),
            pltpu.VMEM_SHARED((ACC_ROWS, CNT_W), jnp.float32),
        ] + [pltpu.SemaphoreType.DMA] * 10,
    )


def _sc_aggregate(*args):
    return _sc_aggregate_kernel()(*args)


_FIN_BLK = 400


def _fin_body(p0_ref, p1_ref, cnt_ref, z_ref, bf_ref, w2_ref,
              b2_ref, out_ref, pooled_ref):
    i = pl.program_id(0)

    @pl.when(i == 0)
    def _():
        pooled_ref[...] = jnp.zeros_like(pooled_ref)

    cnt = cnt_ref[:, 0:1]
    agg = jnp.concatenate([p0_ref[...], p1_ref[...]], axis=-1)
    h = jnp.maximum(agg / jnp.maximum(cnt, 1.0) + z_ref[...], 0.0)
    bcol = bf_ref[...]
    g_lo = jnp.min(bcol).astype(jnp.int32)
    g_hi = jnp.max(bcol).astype(jnp.int32)

    def body(g, carry):
        mask = bcol == g.astype(jnp.float32)
        m = jnp.max(jnp.where(mask, h, 0.0), axis=0, keepdims=True)
        cur = pooled_ref[pl.ds(g, 1), :]
        pooled_ref[pl.ds(g, 1), :] = jnp.maximum(cur, m)
        return carry

    lax.fori_loop(g_lo, g_hi + 1, body, 0)

    @pl.when(i == pl.num_programs(0) - 1)
    def _():
        logits = jnp.dot(pooled_ref[...], w2_ref[...],
                         preferred_element_type=jnp.float32) + b2_ref[...]
        mx = jnp.max(logits, axis=-1, keepdims=True)
        lse = jnp.log(jnp.sum(jnp.exp(logits - mx), axis=-1, keepdims=True)) + mx
        out_ref[...] = logits - lse


def _finalize(p0, p1, cnt, z, batch_f, W2, b2):
    blk = _FIN_BLK
    return pl.pallas_call(
        _fin_body,
        grid=(N_NODES // blk,),
        in_specs=[
            pl.BlockSpec((blk, HALF), lambda i: (i, 0)),
            pl.BlockSpec((blk, HALF), lambda i: (i, 0)),
            pl.BlockSpec((blk, CNT_W), lambda i: (i, 0)),
            pl.BlockSpec((blk, NHID), lambda i: (i, 0)),
            pl.BlockSpec((blk, 1), lambda i: (i, 0)),
            pl.BlockSpec((NHID, N_CLASSES), lambda i: (0, 0)),
            pl.BlockSpec((1, N_CLASSES), lambda i: (0, 0)),
        ],
        out_specs=pl.BlockSpec((N_GRAPHS, N_CLASSES), lambda i: (0, 0)),
        out_shape=jax.ShapeDtypeStruct((N_GRAPHS, N_CLASSES), jnp.float32),
        scratch_shapes=[pltpu.VMEM((N_GRAPHS, NHID), jnp.float32)],
    )(p0, p1, cnt, z, batch_f, W2, b2)


def kernel(x, edge_index, batch, W_l, W_r, b1, W2, b2):
    y0, y1, z = _linear(x, W_l, W_r, b1)
    src_pad = jnp.concatenate(
        [edge_index[0], jnp.zeros((PAD_EDGES,), jnp.int32)])
    dst_pad = jnp.concatenate(
        [edge_index[1], jnp.full((PAD_EDGES,), DUMMY_ROW, jnp.int32)])
    src2d = src_pad.reshape(N_CHUNKS, CHUNK)
    dst2d = dst_pad.reshape(N_CHUNKS, CHUNK)
    zrow = jnp.zeros((ROWS_PER_TILE, HALF), jnp.float32)
    zcnt = jnp.zeros((ROWS_PER_TILE, CNT_W), jnp.float32)
    ones = jnp.ones((CHUNK, CNT_W), jnp.float32)
    p0, p1, cnt = _sc_aggregate(y0, y1, src2d, dst2d, zrow, zcnt, ones)
    batch_f = batch.astype(jnp.float32).reshape(N_NODES, 1)
    return _finalize(p0, p1, cnt, z, batch_f, W2,
                     b2.reshape(1, N_CLASSES))


# speedup vs baseline: 1.0061x; 1.0061x over previous
import functools
import jax
import jax.numpy as jnp
from jax import lax
from jax.experimental import pallas as pl
from jax.experimental.pallas import tpu as pltpu
from jax.experimental.pallas import tpu_sc as plsc

N_NODES = 10000
N_EDGES = 160000
D_FEAT = 256
NHID = 128
N_CLASSES = 2
N_GRAPHS = 128

NC, NS = 2, 16
HALF = NHID // 2
CHUNK = 128
CHUNKS_PER_TILE = 80
N_CHUNKS = NS * CHUNKS_PER_TILE
PAD_EDGES = N_CHUNKS * CHUNK - N_EDGES
ACC_ROWS = 10240
ROWS_PER_TILE = ACC_ROWS // NS
DUMMY_ROW = N_NODES
CNT_W = 8


def _lin_body(x_ref, wl_ref, wr_ref, b1_ref, y0_ref, y1_ref, z_ref):
    x = x_ref[...]
    y = jnp.dot(x, wl_ref[...], preferred_element_type=jnp.float32)
    y0_ref[...] = y[:, :HALF]
    y1_ref[...] = y[:, HALF:]
    z_ref[...] = jnp.dot(x, wr_ref[...], preferred_element_type=jnp.float32) + b1_ref[...]


def _linear(x, W_l, W_r, b1):
    blk = 1000
    return pl.pallas_call(
        _lin_body,
        grid=(N_NODES // blk,),
        in_specs=[
            pl.BlockSpec((blk, D_FEAT), lambda i: (i, 0)),
            pl.BlockSpec((D_FEAT, NHID), lambda i: (0, 0)),
            pl.BlockSpec((D_FEAT, NHID), lambda i: (0, 0)),
            pl.BlockSpec((1, NHID), lambda i: (0, 0)),
        ],
        out_specs=[
            pl.BlockSpec((blk, HALF), lambda i: (i, 0)),
            pl.BlockSpec((blk, HALF), lambda i: (i, 0)),
            pl.BlockSpec((blk, NHID), lambda i: (i, 0)),
        ],
        out_shape=[
            jax.ShapeDtypeStruct((N_NODES, HALF), jnp.float32),
            jax.ShapeDtypeStruct((N_NODES, HALF), jnp.float32),
            jax.ShapeDtypeStruct((N_NODES, NHID), jnp.float32),
        ],
    )(x, W_l, W_r, b1.reshape(1, NHID))


def _sc_body(y0_hbm, y1_hbm, src_hbm, dst_hbm, zrow_hbm, zcnt_hbm, ones_hbm,
             agg0_out, agg1_out, cnt_out,
             src_v, dst_v, rows_a, rows_b, rows_c, rows_d, ones_v,
             acc_sh, cnt_sh,
             gsem_a, gsem_b, gsem_c, gsem_d,
             ssem_a, ssem_b, ssem_c, ssem_d, sem_o, sem_p):
    c = lax.axis_index("c")
    s = lax.axis_index("s")
    r0 = pl.multiple_of(s * ROWS_PER_TILE, 8)

    c0 = pl.multiple_of(s * CHUNKS_PER_TILE, 8)
    prologue = [
        (zrow_hbm, acc_sh.at[pl.ds(r0, ROWS_PER_TILE)]),
        (zcnt_hbm, cnt_sh.at[pl.ds(r0, ROWS_PER_TILE)]),
        (ones_hbm, ones_v),
        (src_hbm.at[pl.ds(c0, CHUNKS_PER_TILE)], src_v),
        (dst_hbm.at[pl.ds(c0, CHUNKS_PER_TILE)], dst_v),
    ]
    for src_ref, dst_ref in prologue:
        pltpu.async_copy(src_ref, dst_ref, sem_p)
    for src_ref, dst_ref in prologue:
        pltpu.make_async_copy(src_ref, dst_ref, sem_p).wait()

    plsc.subcore_barrier()

    def edge_loop(y_hbm, cnt_lo, cnt_hi):
        bufs = (rows_a, rows_b, rows_c, rows_d)
        gsems = (gsem_a, gsem_b, gsem_c, gsem_d)
        ssems = (ssem_a, ssem_b, ssem_c, ssem_d)
        nsuper = CHUNKS_PER_TILE // 4

        def wait_gather(b):
            pltpu.make_async_copy(y_hbm.at[pl.ds(0, CHUNK)], bufs[b],
                                  gsems[b]).wait()

        def wait_scatter(b):
            pltpu.make_async_copy(bufs[b], acc_sh.at[dst_v.at[0]],
                                  ssems[b]).wait()

        def wait_ones():
            pltpu.make_async_copy(ones_v, cnt_sh.at[dst_v.at[0]],
                                  sem_o).wait()

        for b in range(4):
            pltpu.async_copy(y_hbm.at[src_v.at[b]], bufs[b], gsems[b])

        def super_body(j, carry):
            k0 = 4 * j

            @pl.when(jnp.logical_and(j > cnt_lo, j < cnt_hi))
            def _():
                for _b in range(4):
                    wait_ones()

            for b in range(4):
                wait_gather(b)
                pltpu.async_copy(bufs[b], acc_sh.at[dst_v.at[k0 + b]],
                                 ssems[b], add=True)

                @pl.when(jnp.logical_and(j >= cnt_lo, j < cnt_hi))
                def _():
                    pltpu.async_copy(ones_v, cnt_sh.at[dst_v.at[k0 + b]],
                                     sem_o, add=True)

            @pl.when(j < nsuper - 1)
            def _():
                for b in range(4):
                    wait_scatter(b)
                    pltpu.async_copy(y_hbm.at[src_v.at[k0 + 4 + b]],
                                     bufs[b], gsems[b])
            return carry

        lax.fori_loop(0, nsuper, super_body, 0)
        for b in range(4):
            wait_scatter(b)
        if cnt_hi > cnt_lo:
            for _b in range(4):
                wait_ones()

    nsuper = CHUNKS_PER_TILE // 4

    @pl.when(c == 0)
    def _():
        edge_loop(y0_hbm, 0, nsuper)

    @pl.when(c == 1)
    def _():
        edge_loop(y1_hbm, 0, 0)

    plsc.subcore_barrier()

    nvalid = N_NODES - (NS - 1) * ROWS_PER_TILE

    def writeback(nrows):
        @pl.when(c == 0)
        def _():
            pltpu.sync_copy(acc_sh.at[pl.ds(r0, nrows)],
                            agg0_out.at[pl.ds(r0, nrows)])
            pltpu.sync_copy(cnt_sh.at[pl.ds(r0, nrows)],
                            cnt_out.at[pl.ds(r0, nrows)])

        @pl.when(c == 1)
        def _():
            pltpu.sync_copy(acc_sh.at[pl.ds(r0, nrows)],
                            agg1_out.at[pl.ds(r0, nrows)])

    @pl.when(s < NS - 1)
    def _():
        writeback(ROWS_PER_TILE)

    @pl.when(s == NS - 1)
    def _():
        writeback(nvalid)


@functools.cache
def _sc_aggregate_kernel():
    mesh = plsc.VectorSubcoreMesh(
        core_axis_name="c", subcore_axis_name="s",
        num_cores=NC, num_subcores=NS)
    return pl.kernel(
        _sc_body,
        out_type=[
            jax.ShapeDtypeStruct((N_NODES, HALF), jnp.float32),
            jax.ShapeDtypeStruct((N_NODES, HALF), jnp.float32),
            jax.ShapeDtypeStruct((N_NODES, CNT_W), jnp.float32),
        ],
        mesh=mesh,
        compiler_params=pltpu.CompilerParams(use_tc_tiling_on_sc=False),
        scratch_types=[
            pltpu.VMEM((CHUNKS_PER_TILE, CHUNK), jnp.int32),
            pltpu.VMEM((CHUNKS_PER_TILE, CHUNK), jnp.int32),
            pltpu.VMEM((CHUNK, HALF), jnp.float32),
            pltpu.VMEM((CHUNK, HALF), jnp.float32),
            pltpu.VMEM((CHUNK, HALF), jnp.float32),
            pltpu.VMEM((CHUNK, HALF), jnp.float32),
            pltpu.VMEM((CHUNK, CNT_W), jnp.float32),
            pltpu.VMEM_SHARED((ACC_ROWS, HALF), jnp.float32),
            pltpu.VMEM_SHARED((ACC_ROWS, CNT_W), jnp.float32),
        ] + [pltpu.SemaphoreType.DMA] * 10,
    )


def _sc_aggregate(*args):
    return _sc_aggregate_kernel()(*args)


_FIN_BLK = 400


def _fin_body(p0_ref, p1_ref, cnt_ref, z_ref, bf_ref, w2_ref,
              b2_ref, out_ref, pooled_ref):
    i = pl.program_id(0)

    @pl.when(i == 0)
    def _():
        pooled_ref[...] = jnp.zeros_like(pooled_ref)

    cnt = cnt_ref[:, 0:1]
    agg = jnp.concatenate([p0_ref[...], p1_ref[...]], axis=-1)
    h = jnp.maximum(agg / jnp.maximum(cnt, 1.0) + z_ref[...], 0.0)
    bcol = bf_ref[...]
    g_lo = jnp.min(bcol).astype(jnp.int32)
    g_hi = jnp.max(bcol).astype(jnp.int32)

    def body(g, carry):
        mask = bcol == g.astype(jnp.float32)
        m = jnp.max(jnp.where(mask, h, 0.0), axis=0, keepdims=True)
        cur = pooled_ref[pl.ds(g, 1), :]
        pooled_ref[pl.ds(g, 1), :] = jnp.maximum(cur, m)
        return carry

    lax.fori_loop(g_lo, g_hi + 1, body, 0)

    @pl.when(i == pl.num_programs(0) - 1)
    def _():
        logits = jnp.dot(pooled_ref[...], w2_ref[...],
                         preferred_element_type=jnp.float32) + b2_ref[...]
        mx = jnp.max(logits, axis=-1, keepdims=True)
        lse = jnp.log(jnp.sum(jnp.exp(logits - mx), axis=-1, keepdims=True)) + mx
        out_ref[...] = logits - lse


def _finalize(p0, p1, cnt, z, batch_f, W2, b2):
    blk = _FIN_BLK
    return pl.pallas_call(
        _fin_body,
        grid=(N_NODES // blk,),
        in_specs=[
            pl.BlockSpec((blk, HALF), lambda i: (i, 0)),
            pl.BlockSpec((blk, HALF), lambda i: (i, 0)),
            pl.BlockSpec((blk, CNT_W), lambda i: (i, 0)),
            pl.BlockSpec((blk, NHID), lambda i: (i, 0)),
            pl.BlockSpec((blk, 1), lambda i: (i, 0)),
            pl.BlockSpec((NHID, N_CLASSES), lambda i: (0, 0)),
            pl.BlockSpec((1, N_CLASSES), lambda i: (0, 0)),
        ],
        out_specs=pl.BlockSpec((N_GRAPHS, N_CLASSES), lambda i: (0, 0)),
        out_shape=jax.ShapeDtypeStruct((N_GRAPHS, N_CLASSES), jnp.float32),
        scratch_shapes=[pltpu.VMEM((N_GRAPHS, NHID), jnp.float32)],
    )(p0, p1, cnt, z, batch_f, W2, b2)


def kernel(x, edge_index, batch, W_l, W_r, b1, W2, b2):
    y0, y1, z = _linear(x, W_l, W_r, b1)
    src_pad = jnp.concatenate(
        [edge_index[0], jnp.zeros((PAD_EDGES,), jnp.int32)])
    dst_pad = jnp.concatenate(
        [edge_index[1], jnp.full((PAD_EDGES,), DUMMY_ROW, jnp.int32)])
    src2d = src_pad.reshape(N_CHUNKS, CHUNK)
    dst2d = dst_pad.reshape(N_CHUNKS, CHUNK)
    zrow = jnp.zeros((ROWS_PER_TILE, HALF), jnp.float32)
    zcnt = jnp.zeros((ROWS_PER_TILE, CNT_W), jnp.float32)
    ones = jnp.ones((CHUNK, CNT_W), jnp.float32)
    p0, p1, cnt = _sc_aggregate(y0, y1, src2d, dst2d, zrow, zcnt, ones)
    batch_f = batch.astype(jnp.float32).reshape(N_NODES, 1)
    return _finalize(p0, p1, cnt, z, batch_f, W2,
                     b2.reshape(1, N_CLASSES))


# speedup vs baseline: 1.0131x; 1.0070x over previous
import functools
import jax
import jax.numpy as jnp
from jax import lax
from jax.experimental import pallas as pl
from jax.experimental.pallas import tpu as pltpu
from jax.experimental.pallas import tpu_sc as plsc

N_NODES = 10000
N_EDGES = 160000
D_FEAT = 256
NHID = 128
N_CLASSES = 2
N_GRAPHS = 128

NC, NS = 2, 16
HALF = NHID // 2
CHUNK = 128
CHUNKS_PER_TILE = 80
N_CHUNKS = NS * CHUNKS_PER_TILE
PAD_EDGES = N_CHUNKS * CHUNK - N_EDGES
ACC_ROWS = 10240
ROWS_PER_TILE = ACC_ROWS // NS
DUMMY_ROW = N_NODES
CNT_W = 8


def _lin_body(x_ref, wl_ref, wr_ref, b1_ref, y0_ref, y1_ref, z_ref):
    x = x_ref[...]
    y = jnp.dot(x, wl_ref[...], preferred_element_type=jnp.float32)
    y0_ref[...] = y[:, :HALF]
    y1_ref[...] = y[:, HALF:]
    z_ref[...] = jnp.dot(x, wr_ref[...], preferred_element_type=jnp.float32) + b1_ref[...]


def _linear(x, W_l, W_r, b1):
    blk = 1000
    return pl.pallas_call(
        _lin_body,
        grid=(N_NODES // blk,),
        in_specs=[
            pl.BlockSpec((blk, D_FEAT), lambda i: (i, 0)),
            pl.BlockSpec((D_FEAT, NHID), lambda i: (0, 0)),
            pl.BlockSpec((D_FEAT, NHID), lambda i: (0, 0)),
            pl.BlockSpec((1, NHID), lambda i: (0, 0)),
        ],
        out_specs=[
            pl.BlockSpec((blk, HALF), lambda i: (i, 0)),
            pl.BlockSpec((blk, HALF), lambda i: (i, 0)),
            pl.BlockSpec((blk, NHID), lambda i: (i, 0)),
        ],
        out_shape=[
            jax.ShapeDtypeStruct((N_NODES, HALF), jnp.float32),
            jax.ShapeDtypeStruct((N_NODES, HALF), jnp.float32),
            jax.ShapeDtypeStruct((N_NODES, NHID), jnp.float32),
        ],
    )(x, W_l, W_r, b1.reshape(1, NHID))


def _sc_body(y0_hbm, y1_hbm, src_hbm, dst_hbm, zrow_hbm, zcnt_hbm, ones_hbm,
             agg0_out, agg1_out, cnt_out,
             src_v, dst_v, rows_a, rows_b, rows_c, rows_d, ones_v,
             acc_sh, cnt_sh,
             gsem_a, gsem_b, gsem_c, gsem_d,
             ssem_a, ssem_b, ssem_c, ssem_d, sem_o, sem_p):
    c = lax.axis_index("c")
    s = lax.axis_index("s")
    r0 = pl.multiple_of(s * ROWS_PER_TILE, 8)

    c0 = pl.multiple_of(s * CHUNKS_PER_TILE, 8)
    prologue = [
        (zrow_hbm, acc_sh.at[pl.ds(r0, ROWS_PER_TILE)]),
        (zcnt_hbm, cnt_sh.at[pl.ds(r0, ROWS_PER_TILE)]),
        (ones_hbm, ones_v),
        (src_hbm.at[pl.ds(c0, CHUNKS_PER_TILE)], src_v),
        (dst_hbm.at[pl.ds(c0, CHUNKS_PER_TILE)], dst_v),
    ]
    for src_ref, dst_ref in prologue:
        pltpu.async_copy(src_ref, dst_ref, sem_p)
    for src_ref, dst_ref in prologue:
        pltpu.make_async_copy(src_ref, dst_ref, sem_p).wait()

    plsc.subcore_barrier()

    def edge_loop(y_hbm, cnt_lo, cnt_hi):
        bufs = (rows_a, rows_b, rows_c, rows_d)
        gsems = (gsem_a, gsem_b, gsem_c, gsem_d)
        ssems = (ssem_a, ssem_b, ssem_c, ssem_d)
        nsuper = CHUNKS_PER_TILE // 4

        def wait_gather(b):
            pltpu.make_async_copy(y_hbm.at[pl.ds(0, CHUNK)], bufs[b],
                                  gsems[b]).wait()

        def wait_scatter(b):
            pltpu.make_async_copy(bufs[b], acc_sh.at[dst_v.at[0]],
                                  ssems[b]).wait()

        def wait_ones():
            pltpu.make_async_copy(ones_v, cnt_sh.at[dst_v.at[0]],
                                  sem_o).wait()

        for b in range(4):
            pltpu.async_copy(y_hbm.at[src_v.at[b]], bufs[b], gsems[b])

        def super_body(j, carry):
            k0 = 4 * j

            @pl.when(jnp.logical_and(j > cnt_lo, j < cnt_hi))
            def _():
                for _b in range(4):
                    wait_ones()

            for b in range(4):
                wait_gather(b)
                pltpu.async_copy(bufs[b], acc_sh.at[dst_v.at[k0 + b]],
                                 ssems[b], add=True)

                @pl.when(jnp.logical_and(j >= cnt_lo, j < cnt_hi))
                def _():
                    pltpu.async_copy(ones_v, cnt_sh.at[dst_v.at[k0 + b]],
                                     sem_o, add=True)

            @pl.when(j < nsuper - 1)
            def _():
                for b in range(4):
                    wait_scatter(b)
                    pltpu.async_copy(y_hbm.at[src_v.at[k0 + 4 + b]],
                                     bufs[b], gsems[b])
            return carry

        lax.fori_loop(0, nsuper, super_body, 0)
        for b in range(4):
            wait_scatter(b)
        if cnt_hi > cnt_lo:
            for _b in range(4):
                wait_ones()

    nsuper = CHUNKS_PER_TILE // 4

    @pl.when(c == 0)
    def _():
        edge_loop(y0_hbm, 0, 0)

    @pl.when(c == 1)
    def _():
        edge_loop(y1_hbm, 0, nsuper)

    plsc.subcore_barrier()

    nvalid = N_NODES - (NS - 1) * ROWS_PER_TILE

    def writeback(nrows):
        @pl.when(c == 0)
        def _():
            pltpu.sync_copy(acc_sh.at[pl.ds(r0, nrows)],
                            agg0_out.at[pl.ds(r0, nrows)])

        @pl.when(c == 1)
        def _():
            pltpu.sync_copy(acc_sh.at[pl.ds(r0, nrows)],
                            agg1_out.at[pl.ds(r0, nrows)])
            pltpu.sync_copy(cnt_sh.at[pl.ds(r0, nrows)],
                            cnt_out.at[pl.ds(r0, nrows)])

    @pl.when(s < NS - 1)
    def _():
        writeback(ROWS_PER_TILE)

    @pl.when(s == NS - 1)
    def _():
        writeback(nvalid)


@functools.cache
def _sc_aggregate_kernel():
    mesh = plsc.VectorSubcoreMesh(
        core_axis_name="c", subcore_axis_name="s",
        num_cores=NC, num_subcores=NS)
    return pl.kernel(
        _sc_body,
        out_type=[
            jax.ShapeDtypeStruct((N_NODES, HALF), jnp.float32),
            jax.ShapeDtypeStruct((N_NODES, HALF), jnp.float32),
            jax.ShapeDtypeStruct((N_NODES, CNT_W), jnp.float32),
        ],
        mesh=mesh,
        compiler_params=pltpu.CompilerParams(use_tc_tiling_on_sc=False),
        scratch_types=[
            pltpu.VMEM((CHUNKS_PER_TILE, CHUNK), jnp.int32),
            pltpu.VMEM((CHUNKS_PER_TILE, CHUNK), jnp.int32),
            pltpu.VMEM((CHUNK, HALF), jnp.float32),
            pltpu.VMEM((CHUNK, HALF), jnp.float32),
            pltpu.VMEM((CHUNK, HALF), jnp.float32),
            pltpu.VMEM((CHUNK, HALF), jnp.float32),
            pltpu.VMEM((CHUNK, CNT_W), jnp.float32),
            pltpu.VMEM_SHARED((ACC_ROWS, HALF), jnp.float32),
            pltpu.VMEM_SHARED((ACC_ROWS, CNT_W), jnp.float32),
        ] + [pltpu.SemaphoreType.DMA] * 10,
    )


def _sc_aggregate(*args):
    return _sc_aggregate_kernel()(*args)


_FIN_BLK = 400


def _fin_body(p0_ref, p1_ref, cnt_ref, z_ref, bf_ref, w2_ref,
              b2_ref, out_ref, pooled_ref):
    i = pl.program_id(0)

    @pl.when(i == 0)
    def _():
        pooled_ref[...] = jnp.zeros_like(pooled_ref)

    cnt = cnt_ref[:, 0:1]
    agg = jnp.concatenate([p0_ref[...], p1_ref[...]], axis=-1)
    h = jnp.maximum(agg / jnp.maximum(cnt, 1.0) + z_ref[...], 0.0)
    bcol = bf_ref[...]
    g_lo = jnp.min(bcol).astype(jnp.int32)
    g_hi = jnp.max(bcol).astype(jnp.int32)

    def body(g, carry):
        mask = bcol == g.astype(jnp.float32)
        m = jnp.max(jnp.where(mask, h, 0.0), axis=0, keepdims=True)
        cur = pooled_ref[pl.ds(g, 1), :]
        pooled_ref[pl.ds(g, 1), :] = jnp.maximum(cur, m)
        return carry

    lax.fori_loop(g_lo, g_hi + 1, body, 0)

    @pl.when(i == pl.num_programs(0) - 1)
    def _():
        logits = jnp.dot(pooled_ref[...], w2_ref[...],
                         preferred_element_type=jnp.float32) + b2_ref[...]
        mx = jnp.max(logits, axis=-1, keepdims=True)
        lse = jnp.log(jnp.sum(jnp.exp(logits - mx), axis=-1, keepdims=True)) + mx
        out_ref[...] = logits - lse


def _finalize(p0, p1, cnt, z, batch_f, W2, b2):
    blk = _FIN_BLK
    return pl.pallas_call(
        _fin_body,
        grid=(N_NODES // blk,),
        in_specs=[
            pl.BlockSpec((blk, HALF), lambda i: (i, 0)),
            pl.BlockSpec((blk, HALF), lambda i: (i, 0)),
            pl.BlockSpec((blk, CNT_W), lambda i: (i, 0)),
            pl.BlockSpec((blk, NHID), lambda i: (i, 0)),
            pl.BlockSpec((blk, 1), lambda i: (i, 0)),
            pl.BlockSpec((NHID, N_CLASSES), lambda i: (0, 0)),
            pl.BlockSpec((1, N_CLASSES), lambda i: (0, 0)),
        ],
        out_specs=pl.BlockSpec((N_GRAPHS, N_CLASSES), lambda i: (0, 0)),
        out_shape=jax.ShapeDtypeStruct((N_GRAPHS, N_CLASSES), jnp.float32),
        scratch_shapes=[pltpu.VMEM((N_GRAPHS, NHID), jnp.float32)],
    )(p0, p1, cnt, z, batch_f, W2, b2)


def kernel(x, edge_index, batch, W_l, W_r, b1, W2, b2):
    y0, y1, z = _linear(x, W_l, W_r, b1)
    src_pad = jnp.concatenate(
        [edge_index[0], jnp.zeros((PAD_EDGES,), jnp.int32)])
    dst_pad = jnp.concatenate(
        [edge_index[1], jnp.full((PAD_EDGES,), DUMMY_ROW, jnp.int32)])
    src2d = src_pad.reshape(N_CHUNKS, CHUNK)
    dst2d = dst_pad.reshape(N_CHUNKS, CHUNK)
    zrow = jnp.zeros((ROWS_PER_TILE, HALF), jnp.float32)
    zcnt = jnp.zeros((ROWS_PER_TILE, CNT_W), jnp.float32)
    ones = jnp.ones((CHUNK, CNT_W), jnp.float32)
    p0, p1, cnt = _sc_aggregate(y0, y1, src2d, dst2d, zrow, zcnt, ones)
    batch_f = batch.astype(jnp.float32).reshape(N_NODES, 1)
    return _finalize(p0, p1, cnt, z, batch_f, W2,
                     b2.reshape(1, N_CLASSES))
